# Initial kernel scaffold; baseline (speedup 1.0000x reference)
import jax
import jax.numpy as jnp
from jax.experimental import pallas as pl


def kernel(x, edge_index, pred_edge_index, W1, b1, W2, b2, fc_W, fc_b):
    raise NotImplementedError("write your pallas kernel here")


# speedup vs baseline: 3.3458x; 3.3458x over previous
import functools

import jax
import jax.numpy as jnp
from jax import lax
from jax.experimental import pallas as pl
from jax.experimental.pallas import tpu as pltpu
from jax.experimental.pallas import tpu_sc as plsc

NC = 2
NS = 16
NW = NC * NS
L = 16
C = 80
CL = 40
NQ = 4
DEGW = 16


def _blocks_loop(s, nblocks, body):
    kpb = (nblocks + NS - 1) // NS

    def step(kk, carry):
        b = s + kk * NS

        @pl.when(b < nblocks)
        def _():
            body(b)

        return carry

    lax.fori_loop(0, kpb, step, 0)


def _make_deg_kernel(N, E):
    NCH = E // (NS * C)
    NB = N // 8
    mesh = plsc.VectorSubcoreMesh(core_axis_name="c", subcore_axis_name="s")

    @functools.partial(
        pl.kernel,
        out_type=jax.ShapeDtypeStruct((NC, N, DEGW), jnp.float32),
        mesh=mesh,
        compiler_params=pltpu.CompilerParams(use_tc_tiling_on_sc=False),
        scratch_types=[
            pltpu.VMEM((NCH, C), jnp.int32),
            pltpu.VMEM((C, DEGW), jnp.float32),
            pltpu.VMEM((8, DEGW), jnp.float32),
            pltpu.VMEM_SHARED((N, DEGW), jnp.float32),
        ],
    )
    def deg_kernel(ei_ref, deg_out, dst_idx, ones_v, zbuf, deg_sh):
        cc = lax.axis_index("c")
        s = lax.axis_index("s")
        for i in range(C):
            ones_v[i, :] = jnp.ones((L,), jnp.float32)
        for i in range(8):
            zbuf[i, :] = jnp.zeros((L,), jnp.float32)
        _blocks_loop(s, NB,
                     lambda b: pltpu.sync_copy(zbuf, deg_sh.at[pl.ds(b * 8, 8)]))
        pltpu.sync_copy(ei_ref.at[1, s], dst_idx)
        plsc.subcore_barrier()

        def chunk(j, carry):
            pltpu.sync_copy(ones_v, deg_sh.at[dst_idx.at[j]], add=True)
            return carry

        lax.fori_loop(0, NCH, chunk, 0)
        plsc.subcore_barrier()
        _blocks_loop(s, NB,
                     lambda b: pltpu.sync_copy(deg_sh.at[pl.ds(b * 8, 8)],
                                               deg_out.at[cc, pl.ds(b * 8, 8)]))

    return deg_kernel


def _make_agg_kernel(N, E, Q):
    NCH = E // (NS * C)
    NB = N // 8
    mesh = plsc.VectorSubcoreMesh(core_axis_name="c", subcore_axis_name="s")

    @functools.partial(
        pl.kernel,
        out_type=jax.ShapeDtypeStruct((NQ, N, Q), jnp.float32),
        mesh=mesh,
        compiler_params=pltpu.CompilerParams(use_tc_tiling_on_sc=False),
        scratch_types=[
            pltpu.VMEM((NCH, C), jnp.int32),
            pltpu.VMEM((NCH, C), jnp.int32),
            pltpu.VMEM((C, Q), jnp.float32),
            pltpu.VMEM_SHARED((N, Q), jnp.float32),
        ],
    )
    def agg_kernel(y_ref, ei_ref, agg_out, src_idx, dst_idx, rows, agg_sh):
        cc = lax.axis_index("c")
        s = lax.axis_index("s")
        pltpu.sync_copy(ei_ref.at[0, s], src_idx)
        pltpu.sync_copy(ei_ref.at[1, s], dst_idx)
        for q in range(NQ // NC):
            qq = cc * (NQ // NC) + q
            _blocks_loop(
                s, NB,
                lambda b: pltpu.sync_copy(y_ref.at[qq, pl.ds(b * 8, 8)],
                                          agg_sh.at[pl.ds(b * 8, 8)]))
            plsc.subcore_barrier()

            def chunk(j, carry):
                pltpu.sync_copy(y_ref.at[qq].at[src_idx.at[j]], rows)
                pltpu.sync_copy(rows, agg_sh.at[dst_idx.at[j]], add=True)
                return carry

            lax.fori_loop(0, NCH, chunk, 0)
            plsc.subcore_barrier()
            _blocks_loop(
                s, NB,
                lambda b: pltpu.sync_copy(agg_sh.at[pl.ds(b * 8, 8)],
                                          agg_out.at[qq, pl.ds(b * 8, 8)]))
            plsc.subcore_barrier()

    return agg_kernel


def _make_link_kernel(N, EP, H):
    EPW = EP // NW
    NCH = EPW // CL
    KL = H // L
    NG = (CL + L - 1) // L
    RPAD = NG * L
    mesh = plsc.VectorSubcoreMesh(core_axis_name="c", subcore_axis_name="s")

    @functools.partial(
        pl.kernel,
        out_type=jax.ShapeDtypeStruct((EP,), jnp.float32),
        mesh=mesh,
        compiler_params=pltpu.CompilerParams(use_tc_tiling_on_sc=False,
                                             needs_layout_passes=False),
        scratch_types=[
            pltpu.VMEM((NCH, CL), jnp.int32),
            pltpu.VMEM((NCH, CL), jnp.int32),
            pltpu.VMEM((RPAD, H), jnp.float32),
            pltpu.VMEM((RPAD, H), jnp.float32),
            pltpu.VMEM((RPAD,), jnp.float32),
            pltpu.VMEM((L,), jnp.float32),
        ],
    )
    def link_kernel(h_ref, g_ref, pei_ref, fcb_ref, out_ref,
                    sidx, didx, hbuf, gbuf, res, fcb_v):
        cc = lax.axis_index("c")
        s = lax.axis_index("s")
        w = s * NC + cc
        pltpu.sync_copy(pei_ref.at[0, w], sidx)
        pltpu.sync_copy(pei_ref.at[1, w], didx)
        pltpu.sync_copy(fcb_ref, fcb_v)

        def chunk(j, carry):
            pltpu.sync_copy(h_ref.at[sidx.at[j]], hbuf.at[pl.ds(0, CL)])
            pltpu.sync_copy(g_ref.at[didx.at[j]], gbuf.at[pl.ds(0, CL)])
            fcb = fcb_v[...]
            lane = lax.iota(jnp.int32, 16)
            for grp in range(NG):
                resv = jnp.zeros((L,), jnp.float32)
                for e in range(L):
                    ee = grp * L + e
                    acc = hbuf[ee, pl.ds(0, L)] * gbuf[ee, pl.ds(0, L)]
                    for k in range(1, KL):
                        acc = acc + (hbuf[ee, pl.ds(k * L, L)]
                                     * gbuf[ee, pl.ds(k * L, L)])
                    resv = jnp.where(lane == e, jnp.sum(acc), resv)
                v = resv + fcb
                res[pl.ds(grp * L, L)] = 1.0 / (1.0 + jnp.exp(-v))
            pltpu.sync_copy(res.at[pl.ds(0, CL)],
                            out_ref.at[pl.ds(w * EPW + j * CL, CL)])
            return carry

        lax.fori_loop(0, NCH, chunk, 0)

    return link_kernel


def _dinv(deg_ref):
    return lax.rsqrt(deg_ref[0, :, 0:1] + 1.0)


def _mm1_body(x_ref, w_ref, deg_ref, y_ref):
    Q = y_ref.shape[2]
    dinv = _dinv(deg_ref)
    xw = jnp.dot(x_ref[...], w_ref[...], preferred_element_type=jnp.float32)
    y = xw * dinv
    for k in range(NQ):
        y_ref[k, :, :] = y[:, k * Q:(k + 1) * Q]


def _mm2_body(agg_ref, deg_ref, b_ref, w_ref, y_ref):
    Q = y_ref.shape[2]
    dinv = _dinv(deg_ref)
    h = jnp.concatenate([agg_ref[k] for k in range(NQ)], axis=1)
    h = h * dinv + b_ref[...]
    h = jnp.maximum(h, 0.0)
    xw = jnp.dot(h, w_ref[...], preferred_element_type=jnp.float32)
    y = xw * dinv
    for k in range(NQ):
        y_ref[k, :, :] = y[:, k * Q:(k + 1) * Q]


def _mm3_body(agg_ref, deg_ref, b_ref, fcw_ref, h_ref, g_ref):
    dinv = _dinv(deg_ref)
    h = jnp.concatenate([agg_ref[k] for k in range(NQ)], axis=1)
    h = h * dinv + b_ref[...]
    h_ref[...] = h
    g_ref[...] = h * fcw_ref[...]


@jax.jit
def kernel(x, edge_index, pred_edge_index, W1, b1, W2, b2, fc_W, fc_b):
    N, D = x.shape
    H = W1.shape[1]
    Q = H // NQ
    E = edge_index.shape[1]
    EP = pred_edge_index.shape[1]
    BN = 400
    grid = (N // BN,)

    ei_r = edge_index.reshape(2, NS, E // (NS * C), C)
    pei_r = pred_edge_index.reshape(2, NW, EP // (NW * CL), CL)
    fcb16 = jnp.broadcast_to(fc_b, (L,)).astype(jnp.float32)
    b1r = b1.reshape(1, H)
    b2r = b2.reshape(1, H)
    fcwr = fc_W.reshape(1, H)

    deg2 = _make_deg_kernel(N, E)(ei_r)

    y1 = pl.pallas_call(
        _mm1_body,
        grid=grid,
        in_specs=[
            pl.BlockSpec((BN, D), lambda i: (i, 0)),
            pl.BlockSpec((D, H), lambda i: (0, 0)),
            pl.BlockSpec((1, BN, DEGW), lambda i: (0, i, 0)),
        ],
        out_specs=pl.BlockSpec((NQ, BN, Q), lambda i: (0, i, 0)),
        out_shape=jax.ShapeDtypeStruct((NQ, N, Q), jnp.float32),
    )(x, W1, deg2)

    agg_fn = _make_agg_kernel(N, E, Q)
    agg1 = agg_fn(y1, ei_r)

    y2 = pl.pallas_call(
        _mm2_body,
        grid=grid,
        in_specs=[
            pl.BlockSpec((NQ, BN, Q), lambda i: (0, i, 0)),
            pl.BlockSpec((1, BN, DEGW), lambda i: (0, i, 0)),
            pl.BlockSpec((1, H), lambda i: (0, 0)),
            pl.BlockSpec((H, H), lambda i: (0, 0)),
        ],
        out_specs=pl.BlockSpec((NQ, BN, Q), lambda i: (0, i, 0)),
        out_shape=jax.ShapeDtypeStruct((NQ, N, Q), jnp.float32),
    )(agg1, deg2, b1r, W2)

    agg2 = agg_fn(y2, ei_r)

    h2, g = pl.pallas_call(
        _mm3_body,
        grid=grid,
        in_specs=[
            pl.BlockSpec((NQ, BN, Q), lambda i: (0, i, 0)),
            pl.BlockSpec((1, BN, DEGW), lambda i: (0, i, 0)),
            pl.BlockSpec((1, H), lambda i: (0, 0)),
            pl.BlockSpec((1, H), lambda i: (0, 0)),
        ],
        out_specs=[
            pl.BlockSpec((BN, H), lambda i: (i, 0)),
            pl.BlockSpec((BN, H), lambda i: (i, 0)),
        ],
        out_shape=[
            jax.ShapeDtypeStruct((N, H), jnp.float32),
            jax.ShapeDtypeStruct((N, H), jnp.float32),
        ],
    )(agg2, deg2, b2r, fcwr)

    probs = _make_link_kernel(N, EP, H)(h2, g, pei_r, fcb16)
    return probs


# speedup vs baseline: 4.2238x; 1.2624x over previous
import functools

import jax
import jax.numpy as jnp
from jax import lax
from jax.experimental import pallas as pl
from jax.experimental.pallas import tpu as pltpu
from jax.experimental.pallas import tpu_sc as plsc

NC = 2
NS = 16
NW = NC * NS
L = 16
C = 80
CL = 40
NQ = 4
DEGW = 16


def _blocks_loop(s, nblocks, body):
    kpb = (nblocks + NS - 1) // NS

    def step(kk, carry):
        b = s + kk * NS

        @pl.when(b < nblocks)
        def _():
            body(b)

        return carry

    lax.fori_loop(0, kpb, step, 0)


def _make_deg_kernel(N, E):
    NCH = E // (NS * C)
    NB = N // 8
    mesh = plsc.VectorSubcoreMesh(core_axis_name="c", subcore_axis_name="s")

    @functools.partial(
        pl.kernel,
        out_type=jax.ShapeDtypeStruct((NC, N, DEGW), jnp.float32),
        mesh=mesh,
        compiler_params=pltpu.CompilerParams(use_tc_tiling_on_sc=False),
        scratch_types=[
            pltpu.VMEM((NCH, C), jnp.int32),
            pltpu.VMEM((C, DEGW), jnp.float32),
            pltpu.VMEM((8, DEGW), jnp.float32),
            pltpu.VMEM_SHARED((N, DEGW), jnp.float32),
        ],
    )
    def deg_kernel(ei_ref, deg_out, dst_idx, ones_v, zbuf, deg_sh):
        cc = lax.axis_index("c")
        s = lax.axis_index("s")
        for i in range(C):
            ones_v[i, :] = jnp.ones((L,), jnp.float32)
        for i in range(8):
            zbuf[i, :] = jnp.zeros((L,), jnp.float32)
        _blocks_loop(s, NB,
                     lambda b: pltpu.sync_copy(zbuf, deg_sh.at[pl.ds(b * 8, 8)]))
        pltpu.sync_copy(ei_ref.at[1, s], dst_idx)
        plsc.subcore_barrier()

        def chunk(j, carry):
            pltpu.sync_copy(ones_v, deg_sh.at[dst_idx.at[j]], add=True)
            return carry

        lax.fori_loop(0, NCH, chunk, 0)
        plsc.subcore_barrier()
        _blocks_loop(s, NB,
                     lambda b: pltpu.sync_copy(deg_sh.at[pl.ds(b * 8, 8)],
                                               deg_out.at[cc, pl.ds(b * 8, 8)]))

    return deg_kernel


def _make_agg_kernel(N, E, Q):
    NCH = E // (NS * C)
    NB = N // 8
    mesh = plsc.VectorSubcoreMesh(core_axis_name="c", subcore_axis_name="s")

    @functools.partial(
        pl.kernel,
        out_type=jax.ShapeDtypeStruct((NQ, N, Q), jnp.float32),
        mesh=mesh,
        compiler_params=pltpu.CompilerParams(use_tc_tiling_on_sc=False),
        scratch_types=[
            pltpu.VMEM((NCH, C), jnp.int32),
            pltpu.VMEM((NCH, C), jnp.int32),
            pltpu.VMEM((2, C, Q), jnp.float32),
            pltpu.SemaphoreType.DMA((2,)),
            pltpu.VMEM_SHARED((N, Q), jnp.float32),
        ],
    )
    def agg_kernel(y_ref, ei_ref, agg_out, src_idx, dst_idx, rows, semg,
                   agg_sh):
        cc = lax.axis_index("c")
        s = lax.axis_index("s")
        pltpu.sync_copy(ei_ref.at[0, s], src_idx)
        pltpu.sync_copy(ei_ref.at[1, s], dst_idx)
        for q in range(NQ // NC):
            qq = cc * (NQ // NC) + q
            _blocks_loop(
                s, NB,
                lambda b: pltpu.sync_copy(y_ref.at[qq, pl.ds(b * 8, 8)],
                                          agg_sh.at[pl.ds(b * 8, 8)]))
            plsc.subcore_barrier()

            pltpu.async_copy(y_ref.at[qq].at[src_idx.at[0]], rows.at[0],
                             semg.at[0])

            def chunk(j, carry):
                p = lax.rem(j, 2)
                pltpu.make_async_copy(y_ref.at[qq].at[src_idx.at[j]],
                                      rows.at[p], semg.at[p]).wait()

                @pl.when(j + 1 < NCH)
                def _():
                    pltpu.async_copy(y_ref.at[qq].at[src_idx.at[j + 1]],
                                     rows.at[1 - p], semg.at[1 - p])

                pltpu.sync_copy(rows.at[p], agg_sh.at[dst_idx.at[j]],
                                add=True)
                return carry

            lax.fori_loop(0, NCH, chunk, 0)
            plsc.subcore_barrier()
            _blocks_loop(
                s, NB,
                lambda b: pltpu.sync_copy(agg_sh.at[pl.ds(b * 8, 8)],
                                          agg_out.at[qq, pl.ds(b * 8, 8)]))
            plsc.subcore_barrier()

    return agg_kernel


def _make_link_kernel(N, EP, H):
    EPW = EP // NW
    NCH = EPW // CL
    KL = H // L
    NG = (CL + L - 1) // L
    RPAD = NG * L
    mesh = plsc.VectorSubcoreMesh(core_axis_name="c", subcore_axis_name="s")

    @functools.partial(
        pl.kernel,
        out_type=jax.ShapeDtypeStruct((EP,), jnp.float32),
        mesh=mesh,
        compiler_params=pltpu.CompilerParams(use_tc_tiling_on_sc=False,
                                             needs_layout_passes=False),
        scratch_types=[
            pltpu.VMEM((NCH, CL), jnp.int32),
            pltpu.VMEM((NCH, CL), jnp.int32),
            pltpu.VMEM((2, RPAD, H), jnp.float32),
            pltpu.VMEM((2, RPAD, H), jnp.float32),
            pltpu.VMEM((RPAD,), jnp.float32),
            pltpu.VMEM((L,), jnp.float32),
            pltpu.SemaphoreType.DMA((2,)),
            pltpu.SemaphoreType.DMA((2,)),
        ],
    )
    def link_kernel(h_ref, g_ref, pei_ref, fcb_ref, out_ref,
                    sidx, didx, hbuf, gbuf, res, fcb_v, semh, semg):
        cc = lax.axis_index("c")
        s = lax.axis_index("s")
        w = s * NC + cc
        pltpu.sync_copy(pei_ref.at[0, w], sidx)
        pltpu.sync_copy(pei_ref.at[1, w], didx)
        pltpu.sync_copy(fcb_ref, fcb_v)

        def start(j, p):
            pltpu.async_copy(h_ref.at[sidx.at[j]], hbuf.at[p, pl.ds(0, CL)],
                             semh.at[p])
            pltpu.async_copy(g_ref.at[didx.at[j]], gbuf.at[p, pl.ds(0, CL)],
                             semg.at[p])

        start(0, 0)

        def chunk(j, carry):
            p = lax.rem(j, 2)

            @pl.when(j + 1 < NCH)
            def _():
                start(j + 1, 1 - p)

            pltpu.make_async_copy(h_ref.at[sidx.at[j]],
                                  hbuf.at[p, pl.ds(0, CL)], semh.at[p]).wait()
            pltpu.make_async_copy(g_ref.at[didx.at[j]],
                                  gbuf.at[p, pl.ds(0, CL)], semg.at[p]).wait()
            fcb = fcb_v[...]
            lane = lax.iota(jnp.int32, 16)
            for grp in range(NG):
                resv = jnp.zeros((L,), jnp.float32)
                for e in range(L):
                    ee = grp * L + e
                    acc = hbuf[p, ee, pl.ds(0, L)] * gbuf[p, ee, pl.ds(0, L)]
                    for k in range(1, KL):
                        acc = acc + (hbuf[p, ee, pl.ds(k * L, L)]
                                     * gbuf[p, ee, pl.ds(k * L, L)])
                    resv = jnp.where(lane == e, jnp.sum(acc), resv)
                v = resv + fcb
                res[pl.ds(grp * L, L)] = 1.0 / (1.0 + jnp.exp(-v))
            pltpu.sync_copy(res.at[pl.ds(0, CL)],
                            out_ref.at[pl.ds(w * EPW + j * CL, CL)])
            return carry

        lax.fori_loop(0, NCH, chunk, 0)

    return link_kernel


def _dinv(deg_ref):
    return lax.rsqrt(deg_ref[0, :, 0:1] + 1.0)


def _mm1_body(x_ref, w_ref, deg_ref, y_ref):
    Q = y_ref.shape[2]
    dinv = _dinv(deg_ref)
    xw = jnp.dot(x_ref[...], w_ref[...], preferred_element_type=jnp.float32)
    y = xw * dinv
    for k in range(NQ):
        y_ref[k, :, :] = y[:, k * Q:(k + 1) * Q]


def _mm2_body(agg_ref, deg_ref, b_ref, w_ref, y_ref):
    Q = y_ref.shape[2]
    dinv = _dinv(deg_ref)
    h = jnp.concatenate([agg_ref[k] for k in range(NQ)], axis=1)
    h = h * dinv + b_ref[...]
    h = jnp.maximum(h, 0.0)
    xw = jnp.dot(h, w_ref[...], preferred_element_type=jnp.float32)
    y = xw * dinv
    for k in range(NQ):
        y_ref[k, :, :] = y[:, k * Q:(k + 1) * Q]


def _mm3_body(agg_ref, deg_ref, b_ref, fcw_ref, h_ref, g_ref):
    dinv = _dinv(deg_ref)
    h = jnp.concatenate([agg_ref[k] for k in range(NQ)], axis=1)
    h = h * dinv + b_ref[...]
    h_ref[...] = h
    g_ref[...] = h * fcw_ref[...]


@jax.jit
def kernel(x, edge_index, pred_edge_index, W1, b1, W2, b2, fc_W, fc_b):
    N, D = x.shape
    H = W1.shape[1]
    Q = H // NQ
    E = edge_index.shape[1]
    EP = pred_edge_index.shape[1]
    BN = 400
    grid = (N // BN,)

    ei_r = edge_index.reshape(2, NS, E // (NS * C), C)
    pei_r = pred_edge_index.reshape(2, NW, EP // (NW * CL), CL)
    fcb16 = jnp.broadcast_to(fc_b, (L,)).astype(jnp.float32)
    b1r = b1.reshape(1, H)
    b2r = b2.reshape(1, H)
    fcwr = fc_W.reshape(1, H)

    deg2 = _make_deg_kernel(N, E)(ei_r)

    y1 = pl.pallas_call(
        _mm1_body,
        grid=grid,
        in_specs=[
            pl.BlockSpec((BN, D), lambda i: (i, 0)),
            pl.BlockSpec((D, H), lambda i: (0, 0)),
            pl.BlockSpec((1, BN, DEGW), lambda i: (0, i, 0)),
        ],
        out_specs=pl.BlockSpec((NQ, BN, Q), lambda i: (0, i, 0)),
        out_shape=jax.ShapeDtypeStruct((NQ, N, Q), jnp.float32),
    )(x, W1, deg2)

    agg_fn = _make_agg_kernel(N, E, Q)
    agg1 = agg_fn(y1, ei_r)

    y2 = pl.pallas_call(
        _mm2_body,
        grid=grid,
        in_specs=[
            pl.BlockSpec((NQ, BN, Q), lambda i: (0, i, 0)),
            pl.BlockSpec((1, BN, DEGW), lambda i: (0, i, 0)),
            pl.BlockSpec((1, H), lambda i: (0, 0)),
            pl.BlockSpec((H, H), lambda i: (0, 0)),
        ],
        out_specs=pl.BlockSpec((NQ, BN, Q), lambda i: (0, i, 0)),
        out_shape=jax.ShapeDtypeStruct((NQ, N, Q), jnp.float32),
    )(agg1, deg2, b1r, W2)

    agg2 = agg_fn(y2, ei_r)

    h2, g = pl.pallas_call(
        _mm3_body,
        grid=grid,
        in_specs=[
            pl.BlockSpec((NQ, BN, Q), lambda i: (0, i, 0)),
            pl.BlockSpec((1, BN, DEGW), lambda i: (0, i, 0)),
            pl.BlockSpec((1, H), lambda i: (0, 0)),
            pl.BlockSpec((1, H), lambda i: (0, 0)),
        ],
        out_specs=[
            pl.BlockSpec((BN, H), lambda i: (i, 0)),
            pl.BlockSpec((BN, H), lambda i: (i, 0)),
        ],
        out_shape=[
            jax.ShapeDtypeStruct((N, H), jnp.float32),
            jax.ShapeDtypeStruct((N, H), jnp.float32),
        ],
    )(agg2, deg2, b2r, fcwr)

    probs = _make_link_kernel(N, EP, H)(h2, g, pei_r, fcb16)
    return probs


# speedup vs baseline: 8.6072x; 2.0378x over previous
import functools

import jax
import jax.numpy as jnp
from jax import lax
from jax.experimental import pallas as pl
from jax.experimental.pallas import tpu as pltpu
from jax.experimental.pallas import tpu_sc as plsc

NC = 2
NS = 16
NW = NC * NS
L = 16
C = 400
CL = 64
NQ = 4
DEGW = 16


def _span_copy(s, n, copy_fn):
    sz = n // NS
    copy_fn(s * sz, sz)


def _make_deg_kernel(N, E):
    NCH = E // (NS * C)
    NB = N // 8
    mesh = plsc.VectorSubcoreMesh(core_axis_name="c", subcore_axis_name="s")

    @functools.partial(
        pl.kernel,
        out_type=jax.ShapeDtypeStruct((NC, N, DEGW), jnp.float32),
        mesh=mesh,
        compiler_params=pltpu.CompilerParams(use_tc_tiling_on_sc=False),
        scratch_types=[
            pltpu.VMEM((NCH, C), jnp.int32),
            pltpu.VMEM((C, DEGW), jnp.float32),
            pltpu.VMEM((N // NS, DEGW), jnp.float32),
            pltpu.VMEM_SHARED((N, DEGW), jnp.float32),
        ],
    )
    def deg_kernel(ei_ref, deg_out, dst_idx, ones_v, zbuf, deg_sh):
        cc = lax.axis_index("c")
        s = lax.axis_index("s")
        for i in range(C):
            ones_v[i, :] = jnp.ones((L,), jnp.float32)
        NPS = N // NS
        def fill_z(i, carry):
            zbuf[i, :] = jnp.zeros((L,), jnp.float32)
            return carry
        lax.fori_loop(0, NPS, fill_z, 0)
        pltpu.sync_copy(zbuf, deg_sh.at[pl.ds(s * NPS, NPS)])
        pltpu.sync_copy(ei_ref.at[1, s], dst_idx)
        plsc.subcore_barrier()

        half = NCH - NCH // 2

        def chunk(j, carry):
            pltpu.sync_copy(ones_v, deg_sh.at[dst_idx.at[j]], add=True)
            return carry

        lax.fori_loop(cc * half, jnp.minimum(NCH, (cc + 1) * half), chunk, 0)
        plsc.subcore_barrier()
        _span_copy(s, N,
                   lambda off, nr: pltpu.sync_copy(
                       deg_sh.at[pl.ds(off, nr)],
                       deg_out.at[cc, pl.ds(off, nr)]))

    return deg_kernel


def _make_agg_kernel(N, E, Q):
    NCH = E // (NS * C)
    NB = N // 8
    mesh = plsc.VectorSubcoreMesh(core_axis_name="c", subcore_axis_name="s")

    @functools.partial(
        pl.kernel,
        out_type=jax.ShapeDtypeStruct((NQ, N, Q), jnp.float32),
        mesh=mesh,
        compiler_params=pltpu.CompilerParams(use_tc_tiling_on_sc=False),
        scratch_types=[
            pltpu.VMEM((NCH, C), jnp.int32),
            pltpu.VMEM((NCH, C), jnp.int32),
            pltpu.VMEM((2, C, Q), jnp.float32),
            pltpu.SemaphoreType.DMA((2,)),
            pltpu.SemaphoreType.DMA((2,)),
            pltpu.VMEM_SHARED((N, Q), jnp.float32),
        ],
    )
    def agg_kernel(y_ref, ei_ref, agg_out, src_idx, dst_idx, rows, semg,
                   sems, agg_sh):
        cc = lax.axis_index("c")
        s = lax.axis_index("s")
        pltpu.sync_copy(ei_ref.at[0, s], src_idx)
        pltpu.sync_copy(ei_ref.at[1, s], dst_idx)
        for q in range(NQ // NC):
            qq = cc * (NQ // NC) + q
            _span_copy(s, N,
                       lambda off, nr: pltpu.sync_copy(
                           y_ref.at[qq, pl.ds(off, nr)],
                           agg_sh.at[pl.ds(off, nr)]))
            plsc.subcore_barrier()

            pltpu.async_copy(y_ref.at[qq].at[src_idx.at[0]], rows.at[0],
                             semg.at[0])

            def chunk(j, carry):
                p = lax.rem(j, 2)
                pltpu.make_async_copy(y_ref.at[qq].at[src_idx.at[j]],
                                      rows.at[p], semg.at[p]).wait()

                @pl.when(j + 1 < NCH)
                def _():
                    @pl.when(j >= 1)
                    def _():
                        pltpu.make_async_copy(
                            rows.at[1 - p], agg_sh.at[dst_idx.at[j - 1]],
                            sems.at[1 - p]).wait()

                    pltpu.async_copy(y_ref.at[qq].at[src_idx.at[j + 1]],
                                     rows.at[1 - p], semg.at[1 - p])

                pltpu.async_copy(rows.at[p], agg_sh.at[dst_idx.at[j]],
                                 sems.at[p], add=True)
                return carry

            lax.fori_loop(0, NCH, chunk, 0)
            for jj in (NCH - 2, NCH - 1):
                pltpu.make_async_copy(rows.at[jj % 2],
                                      agg_sh.at[dst_idx.at[jj]],
                                      sems.at[jj % 2]).wait()
            plsc.subcore_barrier()
            _span_copy(s, N,
                       lambda off, nr: pltpu.sync_copy(
                           agg_sh.at[pl.ds(off, nr)],
                           agg_out.at[qq, pl.ds(off, nr)]))
            plsc.subcore_barrier()

    return agg_kernel


def _make_link_kernel(N, EP, H):
    EPW = EP // NW
    NCH = -(-EPW // CL)
    EPWP = NCH * CL
    KL = H // L
    NG = CL // L
    RPAD = NG * L
    mesh = plsc.VectorSubcoreMesh(core_axis_name="c", subcore_axis_name="s")

    @functools.partial(
        pl.kernel,
        out_type=jax.ShapeDtypeStruct((NW * EPWP,), jnp.float32),
        mesh=mesh,
        compiler_params=pltpu.CompilerParams(use_tc_tiling_on_sc=False,
                                             needs_layout_passes=False),
        scratch_types=[
            pltpu.VMEM((NCH, CL), jnp.int32),
            pltpu.VMEM((NCH, CL), jnp.int32),
            pltpu.VMEM((2, RPAD, H), jnp.float32),
            pltpu.VMEM((2, RPAD, H), jnp.float32),
            pltpu.VMEM((RPAD,), jnp.float32),
            pltpu.VMEM((L,), jnp.float32),
            pltpu.SemaphoreType.DMA((2,)),
            pltpu.SemaphoreType.DMA((2,)),
        ],
    )
    def link_kernel(h_ref, g_ref, pei_ref, fcb_ref, out_ref,
                    sidx, didx, hbuf, gbuf, res, fcb_v, semh, semg):
        cc = lax.axis_index("c")
        s = lax.axis_index("s")
        w = s * NC + cc
        pltpu.sync_copy(pei_ref.at[0, w], sidx)
        pltpu.sync_copy(pei_ref.at[1, w], didx)
        pltpu.sync_copy(fcb_ref, fcb_v)

        def start(j, p):
            pltpu.async_copy(h_ref.at[sidx.at[j]], hbuf.at[p, pl.ds(0, CL)],
                             semh.at[p])
            pltpu.async_copy(g_ref.at[didx.at[j]], gbuf.at[p, pl.ds(0, CL)],
                             semg.at[p])

        start(0, 0)

        def chunk(j, carry):
            p = lax.rem(j, 2)

            @pl.when(j + 1 < NCH)
            def _():
                start(j + 1, 1 - p)

            pltpu.make_async_copy(h_ref.at[sidx.at[j]],
                                  hbuf.at[p, pl.ds(0, CL)], semh.at[p]).wait()
            pltpu.make_async_copy(g_ref.at[didx.at[j]],
                                  gbuf.at[p, pl.ds(0, CL)], semg.at[p]).wait()
            fcb = fcb_v[...]
            lane = lax.iota(jnp.int32, 16)

            def group(grp, carry):
                resv = jnp.zeros((L,), jnp.float32)
                for e in range(L):
                    ee = grp * L + e
                    acc = hbuf[p, ee, pl.ds(0, L)] * gbuf[p, ee, pl.ds(0, L)]
                    for k in range(1, KL):
                        acc = acc + (hbuf[p, ee, pl.ds(k * L, L)]
                                     * gbuf[p, ee, pl.ds(k * L, L)])
                    resv = jnp.where(lane == e, jnp.sum(acc), resv)
                v = resv + fcb
                res[pl.ds(grp * L, L)] = 1.0 / (1.0 + jnp.exp(-v))
                return carry

            lax.fori_loop(0, NG, group, 0)
            pltpu.sync_copy(res.at[pl.ds(0, CL)],
                            out_ref.at[pl.ds(w * EPWP + j * CL, CL)])
            return carry

        lax.fori_loop(0, NCH, chunk, 0)

    return link_kernel


def _dinv(deg_ref):
    return lax.rsqrt(deg_ref[0, :, 0:1] + deg_ref[1, :, 0:1] + 1.0)


def _mm1_body(x_ref, w_ref, deg_ref, y_ref):
    Q = y_ref.shape[2]
    dinv = _dinv(deg_ref)
    xw = jnp.dot(x_ref[...], w_ref[...], preferred_element_type=jnp.float32)
    y = xw * dinv
    for k in range(NQ):
        y_ref[k, :, :] = y[:, k * Q:(k + 1) * Q]


def _mm2_body(agg_ref, deg_ref, b_ref, w_ref, y_ref):
    Q = y_ref.shape[2]
    dinv = _dinv(deg_ref)
    h = jnp.concatenate([agg_ref[k] for k in range(NQ)], axis=1)
    h = h * dinv + b_ref[...]
    h = jnp.maximum(h, 0.0)
    xw = jnp.dot(h, w_ref[...], preferred_element_type=jnp.float32)
    y = xw * dinv
    for k in range(NQ):
        y_ref[k, :, :] = y[:, k * Q:(k + 1) * Q]


def _mm3_body(agg_ref, deg_ref, b_ref, fcw_ref, h_ref, g_ref):
    dinv = _dinv(deg_ref)
    h = jnp.concatenate([agg_ref[k] for k in range(NQ)], axis=1)
    h = h * dinv + b_ref[...]
    h_ref[...] = h
    g_ref[...] = h * fcw_ref[...]


@jax.jit
def kernel(x, edge_index, pred_edge_index, W1, b1, W2, b2, fc_W, fc_b):
    N, D = x.shape
    H = W1.shape[1]
    Q = H // NQ
    E = edge_index.shape[1]
    EP = pred_edge_index.shape[1]
    BN = 400
    grid = (N // BN,)

    ei_r = edge_index.reshape(2, NS, E // (NS * C), C)
    EPW = EP // NW
    NCHL = -(-EPW // CL)
    EPWP = NCHL * CL
    pei_r = jnp.pad(pred_edge_index.reshape(2, NW, EPW),
                    ((0, 0), (0, 0), (0, EPWP - EPW)))
    pei_r = pei_r.reshape(2, NW, NCHL, CL)
    fcb16 = jnp.broadcast_to(fc_b, (L,)).astype(jnp.float32)
    b1r = b1.reshape(1, H)
    b2r = b2.reshape(1, H)
    fcwr = fc_W.reshape(1, H)

    deg2 = _make_deg_kernel(N, E)(ei_r)

    y1 = pl.pallas_call(
        _mm1_body,
        grid=grid,
        in_specs=[
            pl.BlockSpec((BN, D), lambda i: (i, 0)),
            pl.BlockSpec((D, H), lambda i: (0, 0)),
            pl.BlockSpec((NC, BN, DEGW), lambda i: (0, i, 0)),
        ],
        out_specs=pl.BlockSpec((NQ, BN, Q), lambda i: (0, i, 0)),
        out_shape=jax.ShapeDtypeStruct((NQ, N, Q), jnp.float32),
    )(x, W1, deg2)

    agg_fn = _make_agg_kernel(N, E, Q)
    agg1 = agg_fn(y1, ei_r)

    y2 = pl.pallas_call(
        _mm2_body,
        grid=grid,
        in_specs=[
            pl.BlockSpec((NQ, BN, Q), lambda i: (0, i, 0)),
            pl.BlockSpec((NC, BN, DEGW), lambda i: (0, i, 0)),
            pl.BlockSpec((1, H), lambda i: (0, 0)),
            pl.BlockSpec((H, H), lambda i: (0, 0)),
        ],
        out_specs=pl.BlockSpec((NQ, BN, Q), lambda i: (0, i, 0)),
        out_shape=jax.ShapeDtypeStruct((NQ, N, Q), jnp.float32),
    )(agg1, deg2, b1r, W2)

    agg2 = agg_fn(y2, ei_r)

    h2, g = pl.pallas_call(
        _mm3_body,
        grid=grid,
        in_specs=[
            pl.BlockSpec((NQ, BN, Q), lambda i: (0, i, 0)),
            pl.BlockSpec((NC, BN, DEGW), lambda i: (0, i, 0)),
            pl.BlockSpec((1, H), lambda i: (0, 0)),
            pl.BlockSpec((1, H), lambda i: (0, 0)),
        ],
        out_specs=[
            pl.BlockSpec((BN, H), lambda i: (i, 0)),
            pl.BlockSpec((BN, H), lambda i: (i, 0)),
        ],
        out_shape=[
            jax.ShapeDtypeStruct((N, H), jnp.float32),
            jax.ShapeDtypeStruct((N, H), jnp.float32),
        ],
    )(agg2, deg2, b2r, fcwr)

    probs_p = _make_link_kernel(N, EP, H)(h2, g, pei_r, fcb16)
    probs = probs_p.reshape(NW, EPWP)[:, :EPW].reshape(EP)
    return probs
